# baseline (speedup 1.0000x reference)
import functools

import jax
import jax.numpy as jnp
from jax import lax
from jax.experimental import pallas as pl
from jax.experimental.pallas import tpu as pltpu
from jax.experimental.pallas import tpu_sc as plsc

_B, _C = 16, 96
_NTHETA, _NY, _NX = 8, 64, 64
_ROWS = _B * _C
_IN_ROW = _NTHETA * _NY * _NX
_OUT_ROW = _IN_ROW // 4
_NW = 32
_LINES = _NTHETA * (_NY // 2)

_SC_ROWS = 512
_TC_ROWS = _ROWS - _SC_ROWS
_M_TC = _TC_ROWS * _LINES
_BM = 512

_mesh = plsc.VectorSubcoreMesh(core_axis_name="c", subcore_axis_name="s")


def _make_pool_sc(rows):
    rpw = rows // _NW

    @functools.partial(
        pl.kernel,
        mesh=_mesh,
        out_type=jax.ShapeDtypeStruct((rows, _OUT_ROW), jnp.float32),
        scratch_types=[
            pltpu.VMEM((_IN_ROW,), jnp.float32),
            pltpu.VMEM((_IN_ROW,), jnp.float32),
            pltpu.VMEM((_OUT_ROW,), jnp.float32),
            pltpu.VMEM((_OUT_ROW,), jnp.float32),
            pltpu.SemaphoreType.DMA,
            pltpu.SemaphoreType.DMA,
            pltpu.SemaphoreType.DMA,
            pltpu.SemaphoreType.DMA,
        ],
        compiler_params=pltpu.CompilerParams(needs_layout_passes=False),
    )
    def _pool_sc(x_hbm, out_hbm, in0, in1, o0, o1, si0, si1, so0, so1):
        wid = lax.axis_index("s") * 2 + lax.axis_index("c")
        row0 = wid * rpw
        in_v = (in0, in1)
        out_v = (o0, o1)
        sem_i = (si0, si1)
        sem_o = (so0, so1)
        iota = lax.broadcasted_iota(jnp.int32, (16,), 0)
        idx = [2 * iota + off for off in (0, 1, 64, 65)]

        def compute_row(src, dst):
            def line_body(l, _):
                for g in (0, 1):
                    win = src.at[pl.ds(l * 128 + 32 * g, 96)]
                    v = [plsc.load_gather(win, [e]) for e in idx]
                    dst[pl.ds(l * 32 + 16 * g, 16)] = (
                        (v[0] + v[1]) + (v[2] + v[3])
                    ) * 0.25
                return 0

            lax.fori_loop(0, _LINES, line_body, 0, unroll=8)

        pltpu.async_copy(x_hbm.at[row0], in0, si0)

        def pair_body(ii, _):
            for b in (0, 1):
                i = 2 * ii + b
                r = row0 + i
                if b == 0:
                    pltpu.async_copy(x_hbm.at[r + 1], in_v[1], sem_i[1])
                else:
                    @pl.when(ii < rpw // 2 - 1)
                    def _():
                        pltpu.async_copy(x_hbm.at[r + 1], in_v[0], sem_i[0])

                pltpu.make_async_copy(x_hbm.at[row0], in_v[b], sem_i[b]).wait()
                @pl.when(ii >= 1)
                def _():
                    pltpu.make_async_copy(
                        out_v[b], out_hbm.at[row0], sem_o[b]
                    ).wait()

                compute_row(in_v[b], out_v[b])
                pltpu.async_copy(out_v[b], out_hbm.at[r], sem_o[b])
            return 0

        lax.fori_loop(0, rpw // 2, pair_body, 0)
        for b in (0, 1):
            pltpu.make_async_copy(out_v[b], out_hbm.at[row0], sem_o[b]).wait()

    return _pool_sc


_pool_sc = _make_pool_sc(_SC_ROWS)


def _tc_body(x_ref, p_ref, o_ref):
    a = x_ref[...]
    ysum = a[:, :_NX] + a[:, _NX:]
    o_ref[...] = jnp.dot(
        ysum,
        p_ref[...],
        precision=lax.Precision.HIGHEST,
        preferred_element_type=jnp.float32,
    )


_pool_tc = pl.pallas_call(
    _tc_body,
    grid=(_M_TC // _BM,),
    in_specs=[
        pl.BlockSpec((_BM, 2 * _NX), lambda i: (i, 0)),
        pl.BlockSpec((_NX, _NX // 2), lambda i: (0, 0)),
    ],
    out_specs=pl.BlockSpec((_BM, _NX // 2), lambda i: (i, 0)),
    out_shape=jax.ShapeDtypeStruct((_M_TC, _NX // 2), jnp.float32),
)


def kernel(x):
    xr = x.reshape(_ROWS, _IN_ROW)
    out_sc = _pool_sc(xr[:_SC_ROWS])
    ii = lax.broadcasted_iota(jnp.int32, (_NX, _NX // 2), 0)
    jj = lax.broadcasted_iota(jnp.int32, (_NX, _NX // 2), 1)
    p = jnp.where(ii // 2 == jj, jnp.float32(0.25), jnp.float32(0.0))
    x_tc = xr[_SC_ROWS:].reshape(_M_TC, 2 * _NX)
    out_tc = _pool_tc(x_tc, p).reshape(_TC_ROWS, _OUT_ROW)
    out = jnp.concatenate([out_sc, out_tc], axis=0)
    return out.reshape(_B, _C, _OUT_ROW)


# speedup vs baseline: 1.3429x; 1.3429x over previous
import functools

import jax
import jax.numpy as jnp
from jax import lax
from jax.experimental import pallas as pl
from jax.experimental.pallas import tpu as pltpu
from jax.experimental.pallas import tpu_sc as plsc

_B, _C = 16, 96
_NTHETA, _NY, _NX = 8, 64, 64
_ROWS = _B * _C
_IN_ROW = _NTHETA * _NY * _NX
_OUT_ROW = _IN_ROW // 4
_NW = 32
_LINES = _NTHETA * (_NY // 2)

_SC_ROWS = 512
_TC_ROWS = _ROWS - _SC_ROWS
_M_TC = _TC_ROWS * _LINES
_BM = 2048

_mesh = plsc.VectorSubcoreMesh(core_axis_name="c", subcore_axis_name="s")


def _make_pool_sc(rows):
    rpw = rows // _NW

    @functools.partial(
        pl.kernel,
        mesh=_mesh,
        out_type=jax.ShapeDtypeStruct((rows, _OUT_ROW), jnp.float32),
        scratch_types=[
            pltpu.VMEM((_IN_ROW,), jnp.float32),
            pltpu.VMEM((_IN_ROW,), jnp.float32),
            pltpu.VMEM((_OUT_ROW,), jnp.float32),
            pltpu.VMEM((_OUT_ROW,), jnp.float32),
            pltpu.SemaphoreType.DMA,
            pltpu.SemaphoreType.DMA,
            pltpu.SemaphoreType.DMA,
            pltpu.SemaphoreType.DMA,
        ],
        compiler_params=pltpu.CompilerParams(needs_layout_passes=False),
    )
    def _pool_sc(x_hbm, out_hbm, in0, in1, o0, o1, si0, si1, so0, so1):
        wid = lax.axis_index("s") * 2 + lax.axis_index("c")
        row0 = wid * rpw
        in_v = (in0, in1)
        out_v = (o0, o1)
        sem_i = (si0, si1)
        sem_o = (so0, so1)
        iota = lax.broadcasted_iota(jnp.int32, (16,), 0)
        idx = [2 * iota + off for off in (0, 1, 64, 65)]

        def compute_row(src, dst):
            def line_body(l, _):
                for g in (0, 1):
                    win = src.at[pl.ds(l * 128 + 32 * g, 96)]
                    v = [plsc.load_gather(win, [e]) for e in idx]
                    dst[pl.ds(l * 32 + 16 * g, 16)] = (
                        (v[0] + v[1]) + (v[2] + v[3])
                    ) * 0.25
                return 0

            lax.fori_loop(0, _LINES, line_body, 0, unroll=8)

        pltpu.async_copy(x_hbm.at[row0], in0, si0)

        def pair_body(ii, _):
            for b in (0, 1):
                i = 2 * ii + b
                r = row0 + i
                if b == 0:
                    pltpu.async_copy(x_hbm.at[r + 1], in_v[1], sem_i[1])
                else:
                    @pl.when(ii < rpw // 2 - 1)
                    def _():
                        pltpu.async_copy(x_hbm.at[r + 1], in_v[0], sem_i[0])

                pltpu.make_async_copy(x_hbm.at[row0], in_v[b], sem_i[b]).wait()
                @pl.when(ii >= 1)
                def _():
                    pltpu.make_async_copy(
                        out_v[b], out_hbm.at[row0], sem_o[b]
                    ).wait()

                compute_row(in_v[b], out_v[b])
                pltpu.async_copy(out_v[b], out_hbm.at[r], sem_o[b])
            return 0

        lax.fori_loop(0, rpw // 2, pair_body, 0)
        for b in (0, 1):
            pltpu.make_async_copy(out_v[b], out_hbm.at[row0], sem_o[b]).wait()

    return _pool_sc


_pool_sc = _make_pool_sc(_SC_ROWS)


def _tc_body(x_ref, p_ref, o_ref):
    a = x_ref[...]
    ysum = a[:, :_NX] + a[:, _NX:]
    o_ref[...] = jnp.dot(
        ysum,
        p_ref[...],
        precision=lax.Precision.HIGHEST,
        preferred_element_type=jnp.float32,
    )


_pool_tc = pl.pallas_call(
    _tc_body,
    grid=(_M_TC // _BM,),
    in_specs=[
        pl.BlockSpec((_BM, 2 * _NX), lambda i: (i, 0)),
        pl.BlockSpec((_NX, _NX // 2), lambda i: (0, 0)),
    ],
    out_specs=pl.BlockSpec((_BM, _NX // 2), lambda i: (i, 0)),
    out_shape=jax.ShapeDtypeStruct((_M_TC, _NX // 2), jnp.float32),
)


def kernel(x):
    xr = x.reshape(_ROWS, _IN_ROW)
    out_sc = _pool_sc(xr[:_SC_ROWS])
    ii = lax.broadcasted_iota(jnp.int32, (_NX, _NX // 2), 0)
    jj = lax.broadcasted_iota(jnp.int32, (_NX, _NX // 2), 1)
    p = jnp.where(ii // 2 == jj, jnp.float32(0.25), jnp.float32(0.0))
    x_tc = xr[_SC_ROWS:].reshape(_M_TC, 2 * _NX)
    out_tc = _pool_tc(x_tc, p).reshape(_TC_ROWS, _OUT_ROW)
    out = jnp.concatenate([out_sc, out_tc], axis=0)
    return out.reshape(_B, _C, _OUT_ROW)


# speedup vs baseline: 1.4033x; 1.0450x over previous
import functools

import jax
import jax.numpy as jnp
from jax import lax
from jax.experimental import pallas as pl
from jax.experimental.pallas import tpu as pltpu
from jax.experimental.pallas import tpu_sc as plsc

_B, _C = 16, 96
_NTHETA, _NY, _NX = 8, 64, 64
_ROWS = _B * _C
_IN_ROW = _NTHETA * _NY * _NX
_OUT_ROW = _IN_ROW // 4
_NW = 32
_LINES = _NTHETA * (_NY // 2)

_SC_ROWS = 512
_TC_ROWS = _ROWS - _SC_ROWS
_M_TC = _TC_ROWS * _LINES
_BM = 2048

_mesh = plsc.VectorSubcoreMesh(core_axis_name="c", subcore_axis_name="s")


def _make_pool_sc(rows):
    rpw = rows // _NW

    @functools.partial(
        pl.kernel,
        mesh=_mesh,
        out_type=jax.ShapeDtypeStruct((rows, _OUT_ROW), jnp.float32),
        scratch_types=[
            pltpu.VMEM((_IN_ROW,), jnp.float32),
            pltpu.VMEM((_IN_ROW,), jnp.float32),
            pltpu.VMEM((_OUT_ROW,), jnp.float32),
            pltpu.VMEM((_OUT_ROW,), jnp.float32),
            pltpu.SemaphoreType.DMA,
            pltpu.SemaphoreType.DMA,
            pltpu.SemaphoreType.DMA,
            pltpu.SemaphoreType.DMA,
        ],
        compiler_params=pltpu.CompilerParams(needs_layout_passes=False),
    )
    def _pool_sc(x_hbm, out_hbm, in0, in1, o0, o1, si0, si1, so0, so1):
        wid = lax.axis_index("s") * 2 + lax.axis_index("c")
        row0 = wid * rpw
        in_v = (in0, in1)
        out_v = (o0, o1)
        sem_i = (si0, si1)
        sem_o = (so0, so1)
        iota = lax.broadcasted_iota(jnp.int32, (16,), 0)
        idx = [2 * iota + off for off in (0, 1, 64, 65)]

        def compute_row(src, dst):
            def line_body(l, _):
                for g in (0, 1):
                    win = src.at[pl.ds(l * 128 + 32 * g, 96)]
                    v = [plsc.load_gather(win, [e]) for e in idx]
                    dst[pl.ds(l * 32 + 16 * g, 16)] = (
                        (v[0] + v[1]) + (v[2] + v[3])
                    ) * 0.25
                return 0

            lax.fori_loop(0, _LINES, line_body, 0, unroll=8)

        pltpu.async_copy(x_hbm.at[row0], in0, si0)

        def pair_body(ii, _):
            for b in (0, 1):
                i = 2 * ii + b
                r = row0 + i
                if b == 0:
                    pltpu.async_copy(x_hbm.at[r + 1], in_v[1], sem_i[1])
                else:
                    @pl.when(ii < rpw // 2 - 1)
                    def _():
                        pltpu.async_copy(x_hbm.at[r + 1], in_v[0], sem_i[0])

                pltpu.make_async_copy(x_hbm.at[row0], in_v[b], sem_i[b]).wait()
                @pl.when(ii >= 1)
                def _():
                    pltpu.make_async_copy(
                        out_v[b], out_hbm.at[row0], sem_o[b]
                    ).wait()

                compute_row(in_v[b], out_v[b])
                pltpu.async_copy(out_v[b], out_hbm.at[r], sem_o[b])
            return 0

        lax.fori_loop(0, rpw // 2, pair_body, 0)
        for b in (0, 1):
            pltpu.make_async_copy(out_v[b], out_hbm.at[row0], sem_o[b]).wait()

    return _pool_sc


_pool_sc = _make_pool_sc(_SC_ROWS)


def _tc_body(x_ref, p_ref, o_ref):
    a = x_ref[...]
    ysum = a[:, :_NX] + a[:, _NX:]
    o_ref[...] = jnp.dot(
        ysum,
        p_ref[...],
        precision=lax.Precision.HIGHEST,
        preferred_element_type=jnp.float32,
    )


_TC_OFF = _SC_ROWS * _LINES // _BM

_pool_tc = pl.pallas_call(
    _tc_body,
    grid=(_M_TC // _BM,),
    in_specs=[
        pl.BlockSpec((_BM, 2 * _NX), lambda i: (i + _TC_OFF, 0)),
        pl.BlockSpec((_NX, _NX // 2), lambda i: (0, 0)),
    ],
    out_specs=pl.BlockSpec((_BM, _NX // 2), lambda i: (i, 0)),
    out_shape=jax.ShapeDtypeStruct((_M_TC, _NX // 2), jnp.float32),
)


def kernel(x):
    xr = x.reshape(_ROWS, _IN_ROW)
    out_sc = _pool_sc(xr[:_SC_ROWS])
    ii = lax.broadcasted_iota(jnp.int32, (_NX, _NX // 2), 0)
    jj = lax.broadcasted_iota(jnp.int32, (_NX, _NX // 2), 1)
    p = jnp.where(ii // 2 == jj, jnp.float32(0.25), jnp.float32(0.0))
    x_view = xr.reshape(_ROWS * _LINES, 2 * _NX)
    out_tc = _pool_tc(x_view, p).reshape(_TC_ROWS, _OUT_ROW)
    out = jnp.concatenate([out_sc, out_tc], axis=0)
    return out.reshape(_B, _C, _OUT_ROW)
